# Initial kernel scaffold; baseline (speedup 1.0000x reference)
import jax
import jax.numpy as jnp
from jax.experimental import pallas as pl


def kernel(input_ids, weight, token_type_embeddings, position_embeddings, ln_gamma, ln_beta):
    raise NotImplementedError("write your pallas kernel here")


# speedup vs baseline: 1.6086x; 1.6086x over previous
import functools

import jax
import jax.numpy as jnp
from jax import lax
from jax.experimental import pallas as pl
from jax.experimental.pallas import tpu as pltpu
from jax.experimental.pallas import tpu_sc as plsc

VOCAB = 30000
EMB = 128
B = 4
S = 2048
EPS = 1e-12

NC = 2
NS = 16
NW = NC * NS
TOK = B * S
TPW = TOK // NW
IDXW = 128
NIDX = TPW // IDXW
LANES = 16
NCH = EMB // LANES


@functools.partial(
    pl.kernel,
    out_type=jax.ShapeDtypeStruct((TOK, EMB), jnp.float32),
    mesh=plsc.VectorSubcoreMesh(core_axis_name="c", subcore_axis_name="s"),
    scratch_types=[
        pltpu.VMEM((NIDX, IDXW), jnp.int32),
        pltpu.VMEM((TPW, EMB), jnp.float32),
        pltpu.VMEM((TPW, EMB), jnp.float32),
        pltpu.VMEM((EMB,), jnp.float32),
        pltpu.VMEM((EMB,), jnp.float32),
        pltpu.VMEM((EMB,), jnp.float32),
        pltpu.SemaphoreType.DMA,
        pltpu.SemaphoreType.DMA,
    ],
)
def _emb_ln(ids_hbm, w_hbm, tte_hbm, pos_hbm, g_hbm, beta_hbm, out_hbm,
            idx_v, rows_v, pos_v, g_v, b_v, tte_v, gsem, psem):
    cid = lax.axis_index("c")
    sid = lax.axis_index("s")
    wid = sid * NC + cid
    base = wid * TPW
    pos_base = lax.rem(base, S)

    pltpu.sync_copy(ids_hbm.at[pl.ds(wid * NIDX, NIDX)], idx_v)

    cps = [
        pltpu.async_copy(w_hbm.at[idx_v.at[j]],
                         rows_v.at[pl.ds(j * IDXW, IDXW)], gsem)
        for j in range(NIDX)
    ]
    cpp = pltpu.async_copy(pos_hbm.at[pl.ds(pos_base, TPW)], pos_v, psem)

    pltpu.sync_copy(g_hbm, g_v)
    pltpu.sync_copy(beta_hbm, b_v)
    pltpu.sync_copy(tte_hbm.at[0], tte_v)

    for cp in cps:
        cp.wait()
    cpp.wait()

    gs = [g_v[pl.ds(k * LANES, LANES)] for k in range(NCH)]
    bs = [b_v[pl.ds(k * LANES, LANES)] for k in range(NCH)]
    ts = [tte_v[pl.ds(k * LANES, LANES)] for k in range(NCH)]

    lanes = lax.iota(jnp.int32, LANES)
    perms = [jnp.bitwise_xor(lanes, jnp.int32(k)) for k in (8, 4, 2, 1)]

    def lanesum(a):
        for p in perms:
            a = a + a.at[p].get(mode="promise_in_bounds", unique_indices=True)
        return a

    def body(t, carry):
        xs = []
        for k in range(NCH):
            x = (rows_v[t, pl.ds(k * LANES, LANES)]
                 + pos_v[t, pl.ds(k * LANES, LANES)] + ts[k])
            xs.append(x)
        s0 = xs[0]
        q0 = xs[0] * xs[0]
        for k in range(1, NCH):
            s0 = s0 + xs[k]
            q0 = q0 + xs[k] * xs[k]
        mean = lanesum(s0) * (1.0 / EMB)
        var = lanesum(q0) * (1.0 / EMB) - mean * mean
        v = var + EPS
        i = lax.bitcast_convert_type(v, jnp.int32)
        i = jnp.int32(0x5F3759DF) - lax.shift_right_logical(i, 1)
        y = lax.bitcast_convert_type(i, jnp.float32)
        for _ in range(3):
            y = y * (1.5 - 0.5 * v * y * y)
        for k in range(NCH):
            rows_v[t, pl.ds(k * LANES, LANES)] = (xs[k] - mean) * y * gs[k] + bs[k]
        return carry

    lax.fori_loop(0, TPW, body, 0)
    pltpu.sync_copy(rows_v, out_hbm.at[pl.ds(base, TPW)])


def kernel(input_ids, weight, token_type_embeddings, position_embeddings,
           ln_gamma, ln_beta):
    ids = input_ids.astype(jnp.int32).reshape(TOK // IDXW, IDXW)
    out = _emb_ln(ids, weight, token_type_embeddings, position_embeddings,
                  ln_gamma, ln_beta)
    return out.reshape(B, S, EMB)


# speedup vs baseline: 1.8919x; 1.1761x over previous
import functools

import jax
import jax.numpy as jnp
from jax import lax
from jax.experimental import pallas as pl
from jax.experimental.pallas import tpu as pltpu
from jax.experimental.pallas import tpu_sc as plsc

VOCAB = 30000
EMB = 128
B = 4
S = 2048
EPS = 1e-12

NC = 2
NS = 16
NW = NC * NS
TOK = B * S
TPW = TOK // NW
IDXW = 128
NIDX = TPW // IDXW
LANES = 16
NCH = EMB // LANES


@functools.partial(
    pl.kernel,
    out_type=jax.ShapeDtypeStruct((TOK, EMB), jnp.float32),
    mesh=plsc.VectorSubcoreMesh(core_axis_name="c", subcore_axis_name="s"),
    scratch_types=[
        pltpu.VMEM((NIDX, IDXW), jnp.int32),
        pltpu.VMEM((TPW, EMB), jnp.float32),
        pltpu.VMEM((TPW, EMB), jnp.float32),
        pltpu.VMEM((EMB,), jnp.float32),
        pltpu.VMEM((EMB,), jnp.float32),
        pltpu.VMEM((EMB,), jnp.float32),
        pltpu.SemaphoreType.DMA,
        pltpu.SemaphoreType.DMA,
    ],
)
def _emb_ln(ids_hbm, w_hbm, tte_hbm, pos_hbm, g_hbm, beta_hbm, out_hbm,
            idx_v, rows_v, pos_v, g_v, b_v, tte_v, gsem, psem):
    cid = lax.axis_index("c")
    sid = lax.axis_index("s")
    wid = sid * NC + cid
    base = wid * TPW
    pos_base = lax.rem(base, S)

    pltpu.sync_copy(ids_hbm.at[pl.ds(wid * NIDX, NIDX)], idx_v)

    cps = [
        pltpu.async_copy(w_hbm.at[idx_v.at[j]],
                         rows_v.at[pl.ds(j * IDXW, IDXW)], gsem)
        for j in range(NIDX)
    ]
    cpp = pltpu.async_copy(pos_hbm.at[pl.ds(pos_base, TPW)], pos_v, psem)

    pltpu.sync_copy(g_hbm, g_v)
    pltpu.sync_copy(beta_hbm, b_v)
    pltpu.sync_copy(tte_hbm.at[0], tte_v)

    for cp in cps:
        cp.wait()
    cpp.wait()

    gs = [g_v[pl.ds(k * LANES, LANES)] for k in range(NCH)]
    bs = [b_v[pl.ds(k * LANES, LANES)] for k in range(NCH)]
    ts = [tte_v[pl.ds(k * LANES, LANES)] for k in range(NCH)]

    lanes = lax.iota(jnp.int32, LANES)
    perms = [jnp.bitwise_xor(lanes, jnp.int32(k)) for k in (8, 4, 2, 1)]

    def lanesum(a):
        for p in perms:
            a = a + a.at[p].get(mode="promise_in_bounds", unique_indices=True)
        return a

    UNROLL = 4

    def one_token(t):
        xs = []
        for k in range(NCH):
            x = (rows_v[t, pl.ds(k * LANES, LANES)]
                 + pos_v[t, pl.ds(k * LANES, LANES)] + ts[k])
            xs.append(x)
        s0 = xs[0]
        q0 = xs[0] * xs[0]
        for k in range(1, NCH):
            s0 = s0 + xs[k]
            q0 = q0 + xs[k] * xs[k]
        mean = lanesum(s0) * (1.0 / EMB)
        var = lanesum(q0) * (1.0 / EMB) - mean * mean
        v = var + EPS
        i = lax.bitcast_convert_type(v, jnp.int32)
        i = jnp.int32(0x5F3759DF) - lax.shift_right_logical(i, 1)
        y = lax.bitcast_convert_type(i, jnp.float32)
        for _ in range(3):
            y = y * (1.5 - 0.5 * v * y * y)
        for k in range(NCH):
            rows_v[t, pl.ds(k * LANES, LANES)] = (xs[k] - mean) * y * gs[k] + bs[k]

    def body(i, carry):
        for u in range(UNROLL):
            one_token(i * UNROLL + u)
        return carry

    lax.fori_loop(0, TPW // UNROLL, body, 0)
    pltpu.sync_copy(rows_v, out_hbm.at[pl.ds(base, TPW)])


def kernel(input_ids, weight, token_type_embeddings, position_embeddings,
           ln_gamma, ln_beta):
    ids = input_ids.astype(jnp.int32).reshape(TOK // IDXW, IDXW)
    out = _emb_ln(ids, weight, token_type_embeddings, position_embeddings,
                  ln_gamma, ln_beta)
    return out.reshape(B, S, EMB)
